# baseline (speedup 1.0000x reference)
import functools

import jax
import jax.numpy as jnp
import numpy as np
from jax.experimental import pallas as pl
from jax.experimental.pallas import tpu as pltpu

N_NODES = 10240
N_EDGES = 22528
NUM_FEATURES = 40
EDGE_DIM = 10
HIDDEN = 512
HEADS = 4
N_GRAPHS = 256
HC = HEADS * HIDDEN


def _k1_body(x_ref, w_ref, asd_ref, h2_ref, al_ref):
    j = pl.program_id(1)
    h2 = jnp.dot(x_ref[...], w_ref[...], preferred_element_type=jnp.float32)
    h2_ref[...] = h2
    al = jnp.dot(h2, asd_ref[...], preferred_element_type=jnp.float32)

    @pl.when(j == 0)
    def _():
        al_ref[...] = jnp.zeros_like(al_ref)

    al_ref[...] += al


def _linear_logits(x, wf, asd, bn=512, bc=512):
    n, din = x.shape
    grid = (n // bn, HC // bc)
    return pl.pallas_call(
        _k1_body,
        grid=grid,
        in_specs=[
            pl.BlockSpec((bn, din), lambda i, j: (i, 0)),
            pl.BlockSpec((din, bc), lambda i, j: (0, j)),
            pl.BlockSpec((bc, 2 * HEADS), lambda i, j: (j, 0)),
        ],
        out_specs=[
            pl.BlockSpec((bn, bc), lambda i, j: (i, j)),
            pl.BlockSpec((bn, 2 * HEADS), lambda i, j: (i, 0)),
        ],
        out_shape=[
            jax.ShapeDtypeStruct((n, HC), jnp.float32),
            jax.ShapeDtypeStruct((n, 2 * HEADS), jnp.float32),
        ],
    )(x, wf, asd)


def _gat_layer(x, src, dst, edge_attr, W, We, a_s, a_d, a_e, b):
    H, C = a_s.shape
    wf = W.reshape(W.shape[0], H * C)
    eye = jnp.eye(H, dtype=jnp.float32)
    asd_s = (a_s[:, None, :, None] * eye[:, :, None, None]).transpose(
        0, 2, 1, 3).reshape(H * C, H)
    asd_d = (a_d[:, None, :, None] * eye[:, :, None, None]).transpose(
        0, 2, 1, 3).reshape(H * C, H)
    asd = jnp.concatenate([asd_s, asd_d], axis=1)

    h2, al = _linear_logits(x, wf, asd)
    h = h2.reshape(-1, H, C)
    al_s = al[:, :H]
    al_d = al[:, H:]

    he = (edge_attr @ We.reshape(We.shape[0], H * C)).reshape(-1, H, C)
    al_e = jnp.sum(he * a_e[None], axis=-1)

    e = jax.nn.leaky_relu(al_s[src] + al_d[dst] + al_e, 0.2)
    m = jax.ops.segment_max(e, dst, num_segments=N_NODES)
    m = jnp.where(jnp.isfinite(m), m, 0.0)
    ex = jnp.exp(e - m[dst])
    denom = jax.ops.segment_sum(ex, dst, num_segments=N_NODES)
    alpha = ex / jnp.maximum(denom[dst], 1e-16)
    msg = (h[src] + he) * alpha[:, :, None]
    out = jax.ops.segment_sum(msg, dst, num_segments=N_NODES)
    return jnp.mean(out, axis=1) + b


def kernel(x, edge_index, batch_index, edge_attr, params):
    src = edge_index[0]
    dst = edge_index[1]
    h = x
    for l in range(3):
        h = jax.nn.relu(_gat_layer(
            h, src, dst, edge_attr,
            params['W%d' % l], params['We%d' % l], params['as%d' % l],
            params['ad%d' % l], params['ae%d' % l], params['b%d' % l]))
    mu = jnp.mean(h, axis=0)
    var = jnp.var(h, axis=0)
    h = (h - mu) / jnp.sqrt(var + 1e-5) * params['bn_g'] + params['bn_b']
    gmax = jax.ops.segment_max(h, batch_index, num_segments=N_GRAPHS)
    gmax = jnp.where(jnp.isfinite(gmax), gmax, 0.0)
    counts = jax.ops.segment_sum(
        jnp.ones((h.shape[0], 1), jnp.float32), batch_index,
        num_segments=N_GRAPHS)
    gmean = jax.ops.segment_sum(
        h, batch_index, num_segments=N_GRAPHS) / jnp.maximum(counts, 1.0)
    g = jnp.concatenate([gmax, gmean], axis=1)
    g = jax.nn.relu(g @ params['fc1_W'] + params['fc1_b'])
    g = jax.nn.relu(g @ params['fc2_W'] + params['fc2_b'])
    return g @ params['fc3_W'] + params['fc3_b']
